# baseline (speedup 1.0000x reference)
import functools

import jax
import jax.numpy as jnp
from jax import lax
from jax.experimental import pallas as pl
from jax.experimental.pallas import tpu as pltpu
from jax.experimental.pallas import tpu_sc as plsc

N_CORES = 2
N_SUBCORES = 16
NW = N_CORES * N_SUBCORES


def _sc_aggregate(n_nodes, feats, n_chunks, chunk):
  rows_per_tile = n_nodes // N_SUBCORES

  mesh = plsc.VectorSubcoreMesh(core_axis_name="c", subcore_axis_name="s")

  @functools.partial(
      pl.kernel,
      out_type=(
          jax.ShapeDtypeStruct((N_CORES, n_nodes, feats), jnp.float32),
          jax.ShapeDtypeStruct((N_CORES, n_nodes), jnp.float32),
      ),
      mesh=mesh,
      compiler_params=pltpu.CompilerParams(use_tc_tiling_on_sc=False),
      scratch_types=[
          pltpu.VMEM((n_chunks, chunk), jnp.int32),
          pltpu.VMEM((n_chunks, chunk), jnp.int32),
          pltpu.VMEM((chunk, feats), jnp.float32),
          pltpu.VMEM((chunk, feats), jnp.float32),
          pltpu.VMEM((chunk,), jnp.float32),
          pltpu.VMEM_SHARED((n_nodes, feats), jnp.float32),
          pltpu.VMEM_SHARED((n_nodes,), jnp.float32),
          pltpu.SemaphoreType.DMA,
          pltpu.SemaphoreType.DMA,
          pltpu.SemaphoreType.DMA,
      ],
  )
  def agg(x_hbm, edge_hbm, zacc_hbm, zdeg_hbm, ones_hbm,
          acc_hbm, deg_hbm,
          row_v, col_v, msgs_a, msgs_b, ones_v, acc_sh, deg_sh,
          sem_a, sem_b, sem_s):
    c = lax.axis_index("c")
    s = lax.axis_index("s")
    wid = c * N_SUBCORES + s
    row0 = s * rows_per_tile

    dr = rows_per_tile // 8 * 8
    tail = n_nodes - N_SUBCORES * dr
    pltpu.sync_copy(zacc_hbm, acc_sh.at[pl.ds(row0, rows_per_tile)])
    pltpu.sync_copy(zdeg_hbm, deg_sh.at[pl.ds(s * dr, dr)])
    if tail:
      @pl.when(s == N_SUBCORES - 1)
      def _():
        pltpu.sync_copy(zdeg_hbm.at[pl.ds(0, tail)],
                        deg_sh.at[pl.ds(N_SUBCORES * dr, tail)])
    pltpu.sync_copy(ones_hbm, ones_v)
    pltpu.sync_copy(edge_hbm.at[0, wid], row_v)
    pltpu.sync_copy(edge_hbm.at[1, wid], col_v)

    plsc.subcore_barrier()

    def scatter_chunk(i, buf):
      da = pltpu.async_copy(buf, acc_sh.at[row_v.at[i]], sem_s, add=True)
      dd = pltpu.async_copy(ones_v, deg_sh.at[row_v.at[i]], sem_s, add=True)
      da.wait()
      dd.wait()

    pltpu.async_copy(x_hbm.at[col_v.at[0]], msgs_a, sem_a)

    def body(j, carry):
      i = 2 * j
      pltpu.make_async_copy(x_hbm.at[col_v.at[i]], msgs_a, sem_a).wait()
      pltpu.async_copy(x_hbm.at[col_v.at[i + 1]], msgs_b, sem_b)
      scatter_chunk(i, msgs_a)
      pltpu.make_async_copy(x_hbm.at[col_v.at[i + 1]], msgs_b, sem_b).wait()

      @pl.when(i + 2 < n_chunks)
      def _():
        pltpu.async_copy(x_hbm.at[col_v.at[i + 2]], msgs_a, sem_a)

      scatter_chunk(i + 1, msgs_b)
      return carry
    lax.fori_loop(0, n_chunks // 2, body, 0)

    plsc.subcore_barrier()

    d0 = s * dr
    pltpu.sync_copy(acc_sh.at[pl.ds(d0, dr)], acc_hbm.at[c, pl.ds(d0, dr)])
    pltpu.sync_copy(deg_sh.at[pl.ds(d0, dr)], deg_hbm.at[c, pl.ds(d0, dr)])
    if tail:
      @pl.when(s == N_SUBCORES - 1)
      def _():
        t0 = N_SUBCORES * dr
        pltpu.sync_copy(acc_sh.at[pl.ds(t0, tail)],
                        acc_hbm.at[c, pl.ds(t0, tail)])
        pltpu.sync_copy(deg_sh.at[pl.ds(t0, tail)],
                        deg_hbm.at[c, pl.ds(t0, tail)])

  return agg


def _tc_body(x_ref, acc_ref, deg_ref, w_ref, b_ref, out_ref):
  neigh_sum = acc_ref[0] + acc_ref[1]
  deg = deg_ref[0] + deg_ref[1]
  neigh_mean = neigh_sum / jnp.maximum(deg, 1.0)
  f = x_ref.shape[1]
  out_ref[...] = (
      jnp.dot(x_ref[...], w_ref[0:f], preferred_element_type=jnp.float32)
      + jnp.dot(neigh_mean, w_ref[f : 2 * f],
                preferred_element_type=jnp.float32)
      + b_ref[...]
  )


def kernel(x, edge_index, W, b):
  n, f = x.shape
  e = edge_index.shape[1]
  chunk = 100
  n_chunks = e // (NW * chunk)
  rows_per_tile = n // N_SUBCORES
  edge4 = edge_index.reshape(2, NW, n_chunks, chunk)
  zacc = jnp.zeros((rows_per_tile, f), jnp.float32)
  zdeg = jnp.zeros((rows_per_tile // 8 * 8,), jnp.float32)
  ones = jnp.ones((chunk,), jnp.float32)

  acc, deg = _sc_aggregate(n, f, n_chunks, chunk)(
      x, edge4, zacc, zdeg, ones)
  deg3 = deg.reshape(N_CORES, n, 1)

  mb = 2000
  out = pl.pallas_call(
      _tc_body,
      grid=(n // mb,),
      in_specs=[
          pl.BlockSpec((mb, f), lambda i: (i, 0)),
          pl.BlockSpec((N_CORES, mb, f), lambda i: (0, i, 0)),
          pl.BlockSpec((N_CORES, mb, 1), lambda i: (0, i, 0)),
          pl.BlockSpec((2 * f, f), lambda i: (0, 0)),
          pl.BlockSpec((1, f), lambda i: (0, 0)),
      ],
      out_specs=pl.BlockSpec((mb, f), lambda i: (i, 0)),
      out_shape=jax.ShapeDtypeStruct((n, f), jnp.float32),
  )(x, acc, deg3, W, b.reshape(1, f))
  return out


# speedup vs baseline: 1.0475x; 1.0475x over previous
import functools

import jax
import jax.numpy as jnp
from jax import lax
from jax.experimental import pallas as pl
from jax.experimental.pallas import tpu as pltpu
from jax.experimental.pallas import tpu_sc as plsc

N_CORES = 2
N_SUBCORES = 16
NW = N_CORES * N_SUBCORES
NBUF = 4


def _sc_aggregate(n_nodes, feats, n_chunks, chunk):
  rows_per_tile = n_nodes // N_SUBCORES

  mesh = plsc.VectorSubcoreMesh(core_axis_name="c", subcore_axis_name="s")

  @functools.partial(
      pl.kernel,
      out_type=(
          jax.ShapeDtypeStruct((N_CORES, n_nodes, feats), jnp.float32),
          jax.ShapeDtypeStruct((N_CORES, n_nodes), jnp.float32),
      ),
      mesh=mesh,
      compiler_params=pltpu.CompilerParams(use_tc_tiling_on_sc=False),
      scratch_types=[
          pltpu.VMEM((n_chunks, chunk), jnp.int32),
          pltpu.VMEM((n_chunks, chunk), jnp.int32),
          [pltpu.VMEM((chunk, feats), jnp.float32)] * NBUF,
          pltpu.VMEM((chunk,), jnp.float32),
          pltpu.VMEM_SHARED((n_nodes, feats), jnp.float32),
          pltpu.VMEM_SHARED((n_nodes,), jnp.float32),
          [pltpu.SemaphoreType.DMA] * NBUF,
          pltpu.SemaphoreType.DMA,
      ],
  )
  def agg(x_hbm, edge_hbm, zacc_hbm, zdeg_hbm, ones_hbm,
          acc_hbm, deg_hbm,
          row_v, col_v, msgs, ones_v, acc_sh, deg_sh,
          gsem, sem_s):
    c = lax.axis_index("c")
    s = lax.axis_index("s")
    wid = c * N_SUBCORES + s
    row0 = s * rows_per_tile

    dr = rows_per_tile // 8 * 8
    tail = n_nodes - N_SUBCORES * dr
    pltpu.sync_copy(zacc_hbm, acc_sh.at[pl.ds(row0, rows_per_tile)])
    pltpu.sync_copy(zdeg_hbm, deg_sh.at[pl.ds(s * dr, dr)])
    if tail:
      @pl.when(s == N_SUBCORES - 1)
      def _():
        pltpu.sync_copy(zdeg_hbm.at[pl.ds(0, tail)],
                        deg_sh.at[pl.ds(N_SUBCORES * dr, tail)])
    pltpu.sync_copy(ones_hbm, ones_v)
    pltpu.sync_copy(edge_hbm.at[0, wid], row_v)
    pltpu.sync_copy(edge_hbm.at[1, wid], col_v)

    plsc.subcore_barrier()

    def scatter_chunk(i, buf):
      da = pltpu.async_copy(buf, acc_sh.at[row_v.at[i]], sem_s, add=True)
      dd = pltpu.async_copy(ones_v, deg_sh.at[row_v.at[i]], sem_s, add=True)
      da.wait()
      dd.wait()

    def fire_gather(i, b):
      pltpu.async_copy(x_hbm.at[col_v.at[i]], msgs[b], gsem[b])

    def wait_gather(i, b):
      pltpu.make_async_copy(x_hbm.at[col_v.at[i]], msgs[b], gsem[b]).wait()

    fire_gather(0, 0)
    fire_gather(1, 1)

    def body(j, carry):
      for k in range(NBUF):
        i = NBUF * j + k
        wait_gather(i, k)

        @pl.when(i + 2 < n_chunks)
        def _():
          fire_gather(i + 2, (k + 2) % NBUF)

        scatter_chunk(i, msgs[k])
      return carry
    lax.fori_loop(0, n_chunks // NBUF, body, 0)

    plsc.subcore_barrier()

    d0 = s * dr
    pltpu.sync_copy(acc_sh.at[pl.ds(d0, dr)], acc_hbm.at[c, pl.ds(d0, dr)])
    pltpu.sync_copy(deg_sh.at[pl.ds(d0, dr)], deg_hbm.at[c, pl.ds(d0, dr)])
    if tail:
      @pl.when(s == N_SUBCORES - 1)
      def _():
        t0 = N_SUBCORES * dr
        pltpu.sync_copy(acc_sh.at[pl.ds(t0, tail)],
                        acc_hbm.at[c, pl.ds(t0, tail)])
        pltpu.sync_copy(deg_sh.at[pl.ds(t0, tail)],
                        deg_hbm.at[c, pl.ds(t0, tail)])

  return agg


def _tc_body(x_ref, acc_ref, deg_ref, w_ref, b_ref, out_ref):
  neigh_sum = acc_ref[0] + acc_ref[1]
  deg = deg_ref[0] + deg_ref[1]
  neigh_mean = neigh_sum / jnp.maximum(deg, 1.0)
  f = x_ref.shape[1]
  out_ref[...] = (
      jnp.dot(x_ref[...], w_ref[0:f], preferred_element_type=jnp.float32)
      + jnp.dot(neigh_mean, w_ref[f : 2 * f],
                preferred_element_type=jnp.float32)
      + b_ref[...]
  )


def kernel(x, edge_index, W, b):
  n, f = x.shape
  e = edge_index.shape[1]
  chunk = 50
  n_chunks = e // (NW * chunk)
  rows_per_tile = n // N_SUBCORES
  edge4 = edge_index.reshape(2, NW, n_chunks, chunk)
  zacc = jnp.zeros((rows_per_tile, f), jnp.float32)
  zdeg = jnp.zeros((rows_per_tile // 8 * 8,), jnp.float32)
  ones = jnp.ones((chunk,), jnp.float32)

  acc, deg = _sc_aggregate(n, f, n_chunks, chunk)(
      x, edge4, zacc, zdeg, ones)
  deg3 = deg.reshape(N_CORES, n, 1)

  mb = 2000
  out = pl.pallas_call(
      _tc_body,
      grid=(n // mb,),
      in_specs=[
          pl.BlockSpec((mb, f), lambda i: (i, 0)),
          pl.BlockSpec((N_CORES, mb, f), lambda i: (0, i, 0)),
          pl.BlockSpec((N_CORES, mb, 1), lambda i: (0, i, 0)),
          pl.BlockSpec((2 * f, f), lambda i: (0, 0)),
          pl.BlockSpec((1, f), lambda i: (0, 0)),
      ],
      out_specs=pl.BlockSpec((mb, f), lambda i: (i, 0)),
      out_shape=jax.ShapeDtypeStruct((n, f), jnp.float32),
  )(x, acc, deg3, W, b.reshape(1, f))
  return out


# speedup vs baseline: 1.1818x; 1.1281x over previous
import functools

import jax
import jax.numpy as jnp
from jax import lax
from jax.experimental import pallas as pl
from jax.experimental.pallas import tpu as pltpu
from jax.experimental.pallas import tpu_sc as plsc

N_CORES = 2
N_SUBCORES = 16
NW = N_CORES * N_SUBCORES
NBUF = 5


def _sc_aggregate(n_nodes, feats, n_chunks, chunk):
  rows_per_tile = n_nodes // N_SUBCORES

  mesh = plsc.VectorSubcoreMesh(core_axis_name="c", subcore_axis_name="s")

  @functools.partial(
      pl.kernel,
      out_type=(
          jax.ShapeDtypeStruct((N_CORES, n_nodes, feats), jnp.float32),
          jax.ShapeDtypeStruct((N_CORES, n_nodes), jnp.float32),
      ),
      mesh=mesh,
      compiler_params=pltpu.CompilerParams(use_tc_tiling_on_sc=False),
      scratch_types=[
          pltpu.VMEM((n_chunks * chunk,), jnp.int32),
          pltpu.VMEM((n_chunks * chunk,), jnp.int32),
          [pltpu.VMEM((chunk, feats), jnp.float32)] * NBUF,
          pltpu.VMEM((chunk,), jnp.float32),
          pltpu.VMEM_SHARED((n_nodes, feats), jnp.float32),
          pltpu.VMEM_SHARED((n_nodes,), jnp.float32),
          [pltpu.SemaphoreType.DMA] * NBUF,
          pltpu.SemaphoreType.DMA,
      ],
  )
  def agg(x_hbm, edge_hbm, zacc_hbm, zdeg_hbm, ones_hbm,
          acc_hbm, deg_hbm,
          row_v, col_v, msgs, ones_v, acc_sh, deg_sh,
          gsem, sem_s):
    c = lax.axis_index("c")
    s = lax.axis_index("s")
    wid = c * N_SUBCORES + s
    row0 = s * rows_per_tile

    dr = rows_per_tile // 8 * 8
    tail = n_nodes - N_SUBCORES * dr
    pltpu.sync_copy(zacc_hbm, acc_sh.at[pl.ds(row0, rows_per_tile)])
    pltpu.sync_copy(zdeg_hbm, deg_sh.at[pl.ds(s * dr, dr)])
    if tail:
      @pl.when(s == N_SUBCORES - 1)
      def _():
        pltpu.sync_copy(zdeg_hbm.at[pl.ds(0, tail)],
                        deg_sh.at[pl.ds(N_SUBCORES * dr, tail)])
    pltpu.sync_copy(ones_hbm, ones_v)
    pltpu.sync_copy(edge_hbm.at[0, wid], row_v)
    pltpu.sync_copy(edge_hbm.at[1, wid], col_v)

    plsc.subcore_barrier()

    def scatter_chunk(i, buf):
      ridx = row_v.at[pl.ds(i * chunk, chunk)]
      da = pltpu.async_copy(buf, acc_sh.at[ridx], sem_s, add=True)
      dd = pltpu.async_copy(ones_v, deg_sh.at[ridx], sem_s, add=True)
      da.wait()
      dd.wait()

    def fire_gather(i, b):
      cidx = col_v.at[pl.ds(i * chunk, chunk)]
      pltpu.async_copy(x_hbm.at[cidx], msgs[b], gsem[b])

    def wait_gather(i, b):
      cidx = col_v.at[pl.ds(i * chunk, chunk)]
      pltpu.make_async_copy(x_hbm.at[cidx], msgs[b], gsem[b]).wait()

    fire_gather(0, 0)
    fire_gather(1, 1)

    def body(j, carry):
      for k in range(NBUF):
        i = NBUF * j + k
        wait_gather(i, k)

        @pl.when(i + 2 < n_chunks)
        def _():
          fire_gather(i + 2, (k + 2) % NBUF)

        scatter_chunk(i, msgs[k])
      return carry
    lax.fori_loop(0, n_chunks // NBUF, body, 0)

    plsc.subcore_barrier()

    d0 = s * dr
    pltpu.sync_copy(acc_sh.at[pl.ds(d0, dr)], acc_hbm.at[c, pl.ds(d0, dr)])
    pltpu.sync_copy(deg_sh.at[pl.ds(d0, dr)], deg_hbm.at[c, pl.ds(d0, dr)])
    if tail:
      @pl.when(s == N_SUBCORES - 1)
      def _():
        t0 = N_SUBCORES * dr
        pltpu.sync_copy(acc_sh.at[pl.ds(t0, tail)],
                        acc_hbm.at[c, pl.ds(t0, tail)])
        pltpu.sync_copy(deg_sh.at[pl.ds(t0, tail)],
                        deg_hbm.at[c, pl.ds(t0, tail)])

  return agg


def _tc_body(x_ref, acc_ref, deg_ref, w_ref, b_ref, out_ref):
  neigh_sum = acc_ref[0] + acc_ref[1]
  deg = deg_ref[0] + deg_ref[1]
  neigh_mean = neigh_sum / jnp.maximum(deg, 1.0)[:, None]
  f = x_ref.shape[1]
  out_ref[...] = (
      jnp.dot(x_ref[...], w_ref[0:f], preferred_element_type=jnp.float32)
      + jnp.dot(neigh_mean, w_ref[f : 2 * f],
                preferred_element_type=jnp.float32)
      + b_ref[...]
  )


def kernel(x, edge_index, W, b):
  n, f = x.shape
  e = edge_index.shape[1]
  chunk = 40
  n_chunks = e // (NW * chunk)
  rows_per_tile = n // N_SUBCORES
  edge4 = edge_index.reshape(2, NW, n_chunks * chunk)
  zacc = jnp.zeros((rows_per_tile, f), jnp.float32)
  zdeg = jnp.zeros((rows_per_tile // 8 * 8,), jnp.float32)
  ones = jnp.ones((chunk,), jnp.float32)

  acc, deg = _sc_aggregate(n, f, n_chunks, chunk)(
      x, edge4, zacc, zdeg, ones)

  out = pl.pallas_call(
      _tc_body,
      grid=(1,),
      in_specs=[
          pl.BlockSpec((n, f), lambda i: (0, 0)),
          pl.BlockSpec((N_CORES, n, f), lambda i: (0, 0, 0)),
          pl.BlockSpec((N_CORES, n), lambda i: (0, 0)),
          pl.BlockSpec((2 * f, f), lambda i: (0, 0)),
          pl.BlockSpec((1, f), lambda i: (0, 0)),
      ],
      out_specs=pl.BlockSpec((n, f), lambda i: (0, 0)),
      out_shape=jax.ShapeDtypeStruct((n, f), jnp.float32),
  )(x, acc, deg, W, b.reshape(1, f))
  return out


# speedup vs baseline: 1.2740x; 1.0781x over previous
import functools

import jax
import jax.numpy as jnp
from jax import lax
from jax.experimental import pallas as pl
from jax.experimental.pallas import tpu as pltpu
from jax.experimental.pallas import tpu_sc as plsc

N_CORES = 2
N_SUBCORES = 16
NW = N_CORES * N_SUBCORES
NBUF = 2


def _sc_aggregate(n_nodes, feats, n_chunks, chunk):
  rows_per_tile = n_nodes // N_SUBCORES

  mesh = plsc.VectorSubcoreMesh(core_axis_name="c", subcore_axis_name="s")

  @functools.partial(
      pl.kernel,
      out_type=(
          jax.ShapeDtypeStruct((N_CORES, n_nodes, feats), jnp.float32),
          jax.ShapeDtypeStruct((N_CORES, n_nodes), jnp.float32),
      ),
      mesh=mesh,
      compiler_params=pltpu.CompilerParams(use_tc_tiling_on_sc=False),
      scratch_types=[
          pltpu.VMEM((n_chunks * chunk,), jnp.int32),
          pltpu.VMEM((n_chunks * chunk,), jnp.int32),
          [pltpu.VMEM((chunk, feats), jnp.float32)] * NBUF,
          pltpu.VMEM((chunk,), jnp.float32),
          pltpu.VMEM_SHARED((n_nodes, feats), jnp.float32),
          pltpu.VMEM_SHARED((n_nodes,), jnp.float32),
          [pltpu.SemaphoreType.DMA] * NBUF,
          pltpu.SemaphoreType.DMA,
      ],
  )
  def agg(x_hbm, edge_hbm, zacc_hbm, zdeg_hbm, ones_hbm,
          acc_hbm, deg_hbm,
          row_v, col_v, msgs, ones_v, acc_sh, deg_sh,
          gsem, sem_s):
    c = lax.axis_index("c")
    s = lax.axis_index("s")
    wid = c * N_SUBCORES + s
    row0 = s * rows_per_tile

    dr = rows_per_tile // 8 * 8
    tail = n_nodes - N_SUBCORES * dr
    pltpu.sync_copy(zacc_hbm, acc_sh.at[pl.ds(row0, rows_per_tile)])
    pltpu.sync_copy(zdeg_hbm, deg_sh.at[pl.ds(s * dr, dr)])
    if tail:
      @pl.when(s == N_SUBCORES - 1)
      def _():
        pltpu.sync_copy(zdeg_hbm.at[pl.ds(0, tail)],
                        deg_sh.at[pl.ds(N_SUBCORES * dr, tail)])
    pltpu.sync_copy(ones_hbm, ones_v)
    pltpu.sync_copy(edge_hbm.at[0, wid], row_v)
    pltpu.sync_copy(edge_hbm.at[1, wid], col_v)

    plsc.subcore_barrier()

    def scatter_chunk(i, buf):
      ridx = row_v.at[pl.ds(i * chunk, chunk)]
      da = pltpu.async_copy(buf, acc_sh.at[ridx], sem_s, add=True)
      dd = pltpu.async_copy(ones_v, deg_sh.at[ridx], sem_s, add=True)
      da.wait()
      dd.wait()

    def fire_gather(i, b):
      cidx = col_v.at[pl.ds(i * chunk, chunk)]
      pltpu.async_copy(x_hbm.at[cidx], msgs[b], gsem[b])

    def wait_gather(i, b):
      cidx = col_v.at[pl.ds(i * chunk, chunk)]
      pltpu.make_async_copy(x_hbm.at[cidx], msgs[b], gsem[b]).wait()

    fire_gather(0, 0)
    fire_gather(1, 1)

    def body(j, carry):
      for k in range(NBUF):
        i = NBUF * j + k
        wait_gather(i, k)
        scatter_chunk(i, msgs[k])

        @pl.when(i + 2 < n_chunks)
        def _():
          fire_gather(i + 2, k)
      return carry
    lax.fori_loop(0, n_chunks // NBUF, body, 0)

    for i in range(n_chunks // NBUF * NBUF, n_chunks):
      wait_gather(i, i % NBUF)
      scatter_chunk(i, msgs[i % NBUF])

    plsc.subcore_barrier()

    d0 = s * dr
    pltpu.sync_copy(acc_sh.at[pl.ds(d0, dr)], acc_hbm.at[c, pl.ds(d0, dr)])
    pltpu.sync_copy(deg_sh.at[pl.ds(d0, dr)], deg_hbm.at[c, pl.ds(d0, dr)])
    if tail:
      @pl.when(s == N_SUBCORES - 1)
      def _():
        t0 = N_SUBCORES * dr
        pltpu.sync_copy(acc_sh.at[pl.ds(t0, tail)],
                        acc_hbm.at[c, pl.ds(t0, tail)])
        pltpu.sync_copy(deg_sh.at[pl.ds(t0, tail)],
                        deg_hbm.at[c, pl.ds(t0, tail)])

  return agg


def _tc_body(x_ref, acc_ref, deg_ref, w_ref, b_ref, out_ref):
  neigh_sum = acc_ref[0] + acc_ref[1]
  deg = deg_ref[0] + deg_ref[1]
  neigh_mean = neigh_sum / jnp.maximum(deg, 1.0)[:, None]
  f = x_ref.shape[1]
  out_ref[...] = (
      jnp.dot(x_ref[...], w_ref[0:f], preferred_element_type=jnp.float32)
      + jnp.dot(neigh_mean, w_ref[f : 2 * f],
                preferred_element_type=jnp.float32)
      + b_ref[...]
  )


def kernel(x, edge_index, W, b):
  n, f = x.shape
  e = edge_index.shape[1]
  chunk = 80
  n_chunks = e // (NW * chunk)
  rows_per_tile = n // N_SUBCORES
  edge4 = edge_index.reshape(2, NW, n_chunks * chunk)
  zacc = jnp.zeros((rows_per_tile, f), jnp.float32)
  zdeg = jnp.zeros((rows_per_tile // 8 * 8,), jnp.float32)
  ones = jnp.ones((chunk,), jnp.float32)

  acc, deg = _sc_aggregate(n, f, n_chunks, chunk)(
      x, edge4, zacc, zdeg, ones)

  out = pl.pallas_call(
      _tc_body,
      grid=(1,),
      in_specs=[
          pl.BlockSpec((n, f), lambda i: (0, 0)),
          pl.BlockSpec((N_CORES, n, f), lambda i: (0, 0, 0)),
          pl.BlockSpec((N_CORES, n), lambda i: (0, 0)),
          pl.BlockSpec((2 * f, f), lambda i: (0, 0)),
          pl.BlockSpec((1, f), lambda i: (0, 0)),
      ],
      out_specs=pl.BlockSpec((n, f), lambda i: (0, 0)),
      out_shape=jax.ShapeDtypeStruct((n, f), jnp.float32),
  )(x, acc, deg, W, b.reshape(1, f))
  return out
